# Initial kernel scaffold; baseline (speedup 1.0000x reference)
import jax
import jax.numpy as jnp
from jax.experimental import pallas as pl


def kernel(x, edge_index, W_lin, b_lin, ln_w, ln_b, W_hg, b_hg, gn_w, gn_b, W_skip, b_skip):
    raise NotImplementedError("write your pallas kernel here")


# speedup vs baseline: 6.9982x; 6.9982x over previous
import functools

import jax
import jax.numpy as jnp
from jax import lax
from jax.experimental import pallas as pl
from jax.experimental.pallas import tpu as pltpu
from jax.experimental.pallas import tpu_sc as plsc

NUM_SEGMENTS = 10000
CHUNK = 80
ROWS_BLK = 1000


def _leaky(h):
    return jnp.where(h > 0, h, 0.01 * h)


def _ln(h, w, b):
    mu = jnp.mean(h, axis=1, keepdims=True)
    var = jnp.mean((h - mu) ** 2, axis=1, keepdims=True)
    return (h - mu) * lax.rsqrt(var + 1e-5) * w + b


def _dotT(a, b):
    return lax.dot_general(a, b, (((1,), (1,)), ((), ())),
                           preferred_element_type=jnp.float32)


def _prologue_body(x_ref, wlin_ref, blin_ref, lnw_ref, lnb_ref, whg_ref,
                   wskip_ref, bskip_ref, xw0_ref, xw1_ref, skip_ref):
    h = _dotT(x_ref[...], wlin_ref[...]) + blin_ref[...]
    h = _ln(_leaky(h), lnw_ref[...], lnb_ref[...])
    xw = _dotT(h, whg_ref[...])
    xw0_ref[...] = xw[:, :128]
    xw1_ref[...] = xw[:, 128:]
    skip_ref[...] = _dotT(h, wskip_ref[...]) + bskip_ref[...]


def _scale_body(a0_ref, a1_ref, cnt_ref, o0_ref, o1_ref):
    c = cnt_ref[...][:, 0:1]
    inv = jnp.where(c == 0.0, 0.0, 1.0 / c)
    o0_ref[...] = a0_ref[...] * inv
    o1_ref[...] = a1_ref[...] * inv


def _epilogue_body(a0_ref, a1_ref, cnt_ref, bhg_ref, gnw_ref, gnb_ref,
                   skip_ref, out_ref):
    c = cnt_ref[...][:, 0:1]
    inv = jnp.where(c == 0.0, 0.0, 1.0 / c)
    conv = jnp.concatenate([a0_ref[...], a1_ref[...]], axis=1) * inv
    h = _leaky(conv + bhg_ref[...])
    out_ref[...] = _ln(h, gnw_ref[...], gnb_ref[...]) + skip_ref[...]


def _sub_rows(total, sid_branch):
    base = (total // 16) // 8 * 8
    last = total - 15 * base
    return base, last


def _make_sc_counts(n_nodes, n_edges):
    seg = NUM_SEGMENTS
    per_sub = n_edges // 16
    n_chunks = per_sub // CHUNK
    rb, rl = _sub_rows(max(n_nodes, seg), None)

    @functools.partial(
        pl.kernel,
        mesh=plsc.VectorSubcoreMesh(core_axis_name="c", subcore_axis_name="s"),
        out_type=[
            jax.ShapeDtypeStruct((n_nodes, 128), jnp.float32),
            jax.ShapeDtypeStruct((seg, 128), jnp.float32),
        ],
        scratch_types=[
            pltpu.VMEM_SHARED((max(n_nodes, seg), 128), jnp.float32),
            pltpu.VMEM((CHUNK,), jnp.int32),
            pltpu.VMEM((CHUNK, 128), jnp.float32),
        ],
    )
    def counts(nidx, eidx, zeros_feat, ones_feat,
               dcnt, bcnt, cnt, idx_v, ones_v):
        cid = lax.axis_index("c")
        sid = lax.axis_index("s")

        def each_slice(fn):
            @pl.when(sid < 15)
            def _():
                fn(pl.multiple_of(sid * rb, 8), rb)

            @pl.when(sid == 15)
            def _():
                fn(15 * rb, rl)

        each_slice(lambda r0, sz: pltpu.sync_copy(
            zeros_feat.at[pl.ds(0, sz)], cnt.at[pl.ds(r0, sz)]))
        pltpu.sync_copy(ones_feat, ones_v)
        plsc.subcore_barrier()

        base = sid * per_sub

        def run(src):
            def body(i, carry):
                off = base + i * CHUNK
                pltpu.sync_copy(src.at[pl.ds(off, CHUNK)], idx_v)
                pltpu.sync_copy(ones_v, cnt.at[idx_v], add=True)
                return carry
            lax.fori_loop(0, n_chunks, body, 0)

        @pl.when(cid == 0)
        def _():
            run(nidx)

        @pl.when(cid == 1)
        def _():
            run(eidx)

        plsc.subcore_barrier()

        @pl.when(cid == 0)
        def _():
            each_slice(lambda r0, sz: pltpu.sync_copy(
                cnt.at[pl.ds(r0, sz)], dcnt.at[pl.ds(r0, sz)]))

        @pl.when(cid == 1)
        def _():
            each_slice(lambda r0, sz: pltpu.sync_copy(
                cnt.at[pl.ds(r0, sz)], bcnt.at[pl.ds(r0, sz)]))

    return counts


def _make_sc_stage1(n_nodes, n_edges):
    seg = NUM_SEGMENTS
    per_sub = n_edges // 16
    n_chunks = per_sub // CHUNK
    rb, rl = _sub_rows(seg, None)

    @functools.partial(
        pl.kernel,
        mesh=plsc.VectorSubcoreMesh(core_axis_name="c", subcore_axis_name="s"),
        out_type=[
            jax.ShapeDtypeStruct((seg, 128), jnp.float32),
            jax.ShapeDtypeStruct((seg, 128), jnp.float32),
        ],
        scratch_types=[
            pltpu.VMEM_SHARED((seg, 128), jnp.float32),
            pltpu.VMEM((CHUNK,), jnp.int32),
            pltpu.VMEM((CHUNK,), jnp.int32),
            pltpu.VMEM((CHUNK, 128), jnp.float32),
            pltpu.SemaphoreType.DMA,
        ],
    )
    def stage1(xw0, xw1, nidx, eidx, zeros_feat,
               agg0, agg1, acc, idx_n, idx_e, rows, sem):
        cid = lax.axis_index("c")
        sid = lax.axis_index("s")

        def each_slice(fn):
            @pl.when(sid < 15)
            def _():
                fn(pl.multiple_of(sid * rb, 8), rb)

            @pl.when(sid == 15)
            def _():
                fn(15 * rb, rl)

        each_slice(lambda r0, sz: pltpu.sync_copy(
            zeros_feat.at[pl.ds(0, sz)], acc.at[pl.ds(r0, sz)]))
        plsc.subcore_barrier()

        base = sid * per_sub

        def run(src):
            def body(i, carry):
                off = base + i * CHUNK
                pltpu.sync_copy(nidx.at[pl.ds(off, CHUNK)], idx_n)
                pltpu.sync_copy(eidx.at[pl.ds(off, CHUNK)], idx_e)
                pltpu.async_copy(src.at[idx_n], rows, sem).wait()
                pltpu.sync_copy(rows, acc.at[idx_e], add=True)
                return carry
            lax.fori_loop(0, n_chunks, body, 0)

        @pl.when(cid == 0)
        def _():
            run(xw0)

        @pl.when(cid == 1)
        def _():
            run(xw1)

        plsc.subcore_barrier()

        @pl.when(cid == 0)
        def _():
            each_slice(lambda r0, sz: pltpu.sync_copy(
                acc.at[pl.ds(r0, sz)], agg0.at[pl.ds(r0, sz)]))

        @pl.when(cid == 1)
        def _():
            each_slice(lambda r0, sz: pltpu.sync_copy(
                acc.at[pl.ds(r0, sz)], agg1.at[pl.ds(r0, sz)]))

    return stage1


def _make_sc_stage2(n_nodes, n_edges):
    seg = NUM_SEGMENTS
    per_sub = n_edges // 16
    n_chunks = per_sub // CHUNK
    rb, rl = _sub_rows(n_nodes, None)

    @functools.partial(
        pl.kernel,
        mesh=plsc.VectorSubcoreMesh(core_axis_name="c", subcore_axis_name="s"),
        out_type=[
            jax.ShapeDtypeStruct((n_nodes, 128), jnp.float32),
            jax.ShapeDtypeStruct((n_nodes, 128), jnp.float32),
        ],
        scratch_types=[
            pltpu.VMEM_SHARED((n_nodes, 128), jnp.float32),
            pltpu.VMEM((CHUNK,), jnp.int32),
            pltpu.VMEM((CHUNK,), jnp.int32),
            pltpu.VMEM((CHUNK, 128), jnp.float32),
            pltpu.SemaphoreType.DMA,
        ],
    )
    def stage2(out1_0, out1_1, nidx, eidx, zeros_feat,
               agg2_0, agg2_1, acc, idx_n, idx_e, rows, sem):
        cid = lax.axis_index("c")
        sid = lax.axis_index("s")

        def each_slice(fn):
            @pl.when(sid < 15)
            def _():
                fn(pl.multiple_of(sid * rb, 8), rb)

            @pl.when(sid == 15)
            def _():
                fn(15 * rb, rl)

        each_slice(lambda r0, sz: pltpu.sync_copy(
            zeros_feat.at[pl.ds(0, sz)], acc.at[pl.ds(r0, sz)]))
        plsc.subcore_barrier()

        base = sid * per_sub

        def run(src):
            def body(i, carry):
                off = base + i * CHUNK
                pltpu.sync_copy(nidx.at[pl.ds(off, CHUNK)], idx_n)
                pltpu.sync_copy(eidx.at[pl.ds(off, CHUNK)], idx_e)
                pltpu.async_copy(src.at[idx_e], rows, sem).wait()
                pltpu.sync_copy(rows, acc.at[idx_n], add=True)
                return carry
            lax.fori_loop(0, n_chunks, body, 0)

        @pl.when(cid == 0)
        def _():
            run(out1_0)

        @pl.when(cid == 1)
        def _():
            run(out1_1)

        plsc.subcore_barrier()

        @pl.when(cid == 0)
        def _():
            each_slice(lambda r0, sz: pltpu.sync_copy(
                acc.at[pl.ds(r0, sz)], agg2_0.at[pl.ds(r0, sz)]))

        @pl.when(cid == 1)
        def _():
            each_slice(lambda r0, sz: pltpu.sync_copy(
                acc.at[pl.ds(r0, sz)], agg2_1.at[pl.ds(r0, sz)]))

    return stage2


def kernel(x, edge_index, W_lin, b_lin, ln_w, ln_b, W_hg, b_hg, gn_w, gn_b,
           W_skip, b_skip):
    n, in_c = x.shape
    hid = W_lin.shape[0]
    e = edge_index.shape[1]
    seg = NUM_SEGMENTS
    nidx = edge_index[0]
    eidx = edge_index[1]

    grid = n // ROWS_BLK
    full = lambda shp: pl.BlockSpec(shp, lambda i: (0,) * len(shp))
    row_blk = lambda w: pl.BlockSpec((ROWS_BLK, w), lambda i: (i, 0))

    xw0, xw1, skip = pl.pallas_call(
        _prologue_body,
        grid=(grid,),
        in_specs=[row_blk(in_c), full((hid, in_c)), full((1, hid)),
                  full((1, hid)), full((1, hid)), full((hid, hid)),
                  full((hid, hid)), full((1, hid))],
        out_specs=[row_blk(128), row_blk(128), row_blk(hid)],
        out_shape=[jax.ShapeDtypeStruct((n, 128), jnp.float32),
                   jax.ShapeDtypeStruct((n, 128), jnp.float32),
                   jax.ShapeDtypeStruct((n, hid), jnp.float32)],
    )(x, W_lin, b_lin.reshape(1, hid), ln_w.reshape(1, hid),
      ln_b.reshape(1, hid), W_hg, W_skip, b_skip.reshape(1, hid))

    zrows = max(_sub_rows(seg, None)[1], _sub_rows(n, None)[1])
    zeros_feat = jnp.zeros((zrows, 128), jnp.float32)
    ones_feat = jnp.ones((CHUNK, 128), jnp.float32)

    dcnt, bcnt = _make_sc_counts(n, e)(nidx, eidx, zeros_feat, ones_feat)
    agg0, agg1 = _make_sc_stage1(n, e)(xw0, xw1, nidx, eidx, zeros_feat)

    out1_0, out1_1 = pl.pallas_call(
        _scale_body,
        grid=(seg // ROWS_BLK,),
        in_specs=[row_blk(128), row_blk(128), row_blk(128)],
        out_specs=[row_blk(128), row_blk(128)],
        out_shape=[jax.ShapeDtypeStruct((seg, 128), jnp.float32),
                   jax.ShapeDtypeStruct((seg, 128), jnp.float32)],
    )(agg0, agg1, bcnt)

    agg2_0, agg2_1 = _make_sc_stage2(n, e)(
        out1_0, out1_1, nidx, eidx, zeros_feat)

    out = pl.pallas_call(
        _epilogue_body,
        grid=(grid,),
        in_specs=[row_blk(128), row_blk(128), row_blk(128), full((1, hid)),
                  full((1, hid)), full((1, hid)), row_blk(hid)],
        out_specs=row_blk(hid),
        out_shape=jax.ShapeDtypeStruct((n, hid), jnp.float32),
    )(agg2_0, agg2_1, dcnt, b_hg.reshape(1, hid), gn_w.reshape(1, hid),
      gn_b.reshape(1, hid), skip)

    return out


# speedup vs baseline: 10.2164x; 1.4599x over previous
import functools

import jax
import jax.numpy as jnp
from jax import lax
from jax.experimental import pallas as pl
from jax.experimental.pallas import tpu as pltpu
from jax.experimental.pallas import tpu_sc as plsc

NUM_SEGMENTS = 10000
CHUNK = 128
TRASH = 8
ROWS_BLK = 1000


def _leaky(h):
    return jnp.where(h > 0, h, 0.01 * h)


def _ln(h, w, b):
    mu = jnp.mean(h, axis=1, keepdims=True)
    var = jnp.mean((h - mu) ** 2, axis=1, keepdims=True)
    return (h - mu) * lax.rsqrt(var + 1e-5) * w + b


def _dotT(a, b):
    return lax.dot_general(a, b, (((1,), (1,)), ((), ())),
                           preferred_element_type=jnp.float32)


def _prologue_body(x_ref, wlin_ref, blin_ref, lnw_ref, lnb_ref, whg_ref,
                   wskip_ref, bskip_ref, xw0_ref, xw1_ref, skip_ref):
    h = _dotT(x_ref[...], wlin_ref[...]) + blin_ref[...]
    h = _ln(_leaky(h), lnw_ref[...], lnb_ref[...])
    xw = _dotT(h, whg_ref[...])
    xw0_ref[...] = xw[:, :128]
    xw1_ref[...] = xw[:, 128:]
    skip_ref[...] = _dotT(h, wskip_ref[...]) + bskip_ref[...]


def _scale_body(a0_ref, a1_ref, cnt_ref, o0_ref, o1_ref):
    c = cnt_ref[...][:, 0:1]
    inv = jnp.where(c == 0.0, 0.0, 1.0 / c)
    o0_ref[...] = a0_ref[...] * inv
    o1_ref[...] = a1_ref[...] * inv


def _epilogue_body(a0_ref, a1_ref, cnt_ref, bhg_ref, gnw_ref, gnb_ref,
                   skip_ref, out_ref):
    c = cnt_ref[...][:, 0:1]
    inv = jnp.where(c == 0.0, 0.0, 1.0 / c)
    conv = jnp.concatenate([a0_ref[...], a1_ref[...]], axis=1) * inv
    h = _leaky(conv + bhg_ref[...])
    out_ref[...] = _ln(h, gnw_ref[...], gnb_ref[...]) + skip_ref[...]


def _sub_rows(total):
    base = (total // 16) // 8 * 8
    last = total - 15 * base
    return base, last


def _n_chunks(n_edges):
    per_sub = n_edges // 16
    nch = -(-per_sub // CHUNK)
    return nch + (nch % 2)


def _each_slice(sid, rb, rl, fn):
    @pl.when(sid < 15)
    def _():
        fn(pl.multiple_of(sid * rb, 8), rb)

    @pl.when(sid == 15)
    def _():
        fn(15 * rb, rl)


def _make_sc_counts(n_nodes, n_edges):
    seg = NUM_SEGMENTS
    nch = _n_chunks(n_edges)
    half = nch // 2
    tbl = max(n_nodes, seg) + TRASH
    rb, rl = _sub_rows(max(n_nodes, seg))

    @functools.partial(
        pl.kernel,
        mesh=plsc.VectorSubcoreMesh(core_axis_name="c", subcore_axis_name="s"),
        out_type=[
            jax.ShapeDtypeStruct((n_nodes, 128), jnp.float32),
            jax.ShapeDtypeStruct((seg, 128), jnp.float32),
        ],
        scratch_types=[
            pltpu.VMEM_SHARED((tbl, 128), jnp.float32),
            pltpu.VMEM((2, CHUNK), jnp.int32),
            pltpu.VMEM((2, CHUNK), jnp.int32),
            pltpu.VMEM((CHUNK, 128), jnp.float32),
            pltpu.SemaphoreType.DMA,
            pltpu.SemaphoreType.DMA,
        ],
    )
    def counts(comb1, comb2, zeros_feat, ones_feat,
               dcnt, bcnt, cnt, idx_a, idx_b, ones_v, sem_sa, sem_sb):
        cid = lax.axis_index("c")
        sid = lax.axis_index("s")

        _each_slice(sid, rb, rl, lambda r0, sz: pltpu.sync_copy(
            zeros_feat.at[pl.ds(0, sz)], cnt.at[pl.ds(r0, sz)]))
        pltpu.sync_copy(ones_feat, ones_v)
        plsc.subcore_barrier()

        def run(comb):
            pltpu.sync_copy(comb.at[sid, 0], idx_a)
            pltpu.sync_copy(comb.at[sid, 1], idx_b)

            def body(i, carry):
                c0 = 2 * i
                pltpu.async_copy(ones_v, cnt.at[idx_a.at[1]], sem_sa,
                                 add=True)
                pltpu.async_copy(ones_v, cnt.at[idx_b.at[1]], sem_sb,
                                 add=True)

                @pl.when(i < half - 1)
                def _():
                    pltpu.make_async_copy(ones_v, cnt.at[idx_a.at[1]],
                                          sem_sa).wait()
                    pltpu.sync_copy(comb.at[sid, c0 + 2], idx_a)
                    pltpu.make_async_copy(ones_v, cnt.at[idx_b.at[1]],
                                          sem_sb).wait()
                    pltpu.sync_copy(comb.at[sid, c0 + 3], idx_b)
                return carry

            lax.fori_loop(0, half, body, 0)
            pltpu.make_async_copy(ones_v, cnt.at[idx_a.at[1]], sem_sa).wait()
            pltpu.make_async_copy(ones_v, cnt.at[idx_b.at[1]], sem_sb).wait()

        @pl.when(cid == 0)
        def _():
            run(comb2)

        @pl.when(cid == 1)
        def _():
            run(comb1)

        plsc.subcore_barrier()

        @pl.when(cid == 0)
        def _():
            _each_slice(sid, rb, rl, lambda r0, sz: pltpu.sync_copy(
                cnt.at[pl.ds(r0, sz)], dcnt.at[pl.ds(r0, sz)]))

        @pl.when(cid == 1)
        def _():
            _each_slice(sid, rb, rl, lambda r0, sz: pltpu.sync_copy(
                cnt.at[pl.ds(r0, sz)], bcnt.at[pl.ds(r0, sz)]))

    return counts


def _make_sc_stage(n_rows_out, n_edges):
    nch = _n_chunks(n_edges)
    half = nch // 2
    rb, rl = _sub_rows(n_rows_out)

    @functools.partial(
        pl.kernel,
        mesh=plsc.VectorSubcoreMesh(core_axis_name="c", subcore_axis_name="s"),
        out_type=[
            jax.ShapeDtypeStruct((n_rows_out, 128), jnp.float32),
            jax.ShapeDtypeStruct((n_rows_out, 128), jnp.float32),
        ],
        scratch_types=[
            pltpu.VMEM_SHARED((n_rows_out + TRASH, 128), jnp.float32),
            pltpu.VMEM((2, CHUNK), jnp.int32),
            pltpu.VMEM((2, CHUNK), jnp.int32),
            pltpu.VMEM((CHUNK, 128), jnp.float32),
            pltpu.VMEM((CHUNK, 128), jnp.float32),
            pltpu.SemaphoreType.DMA,
            pltpu.SemaphoreType.DMA,
            pltpu.SemaphoreType.DMA,
            pltpu.SemaphoreType.DMA,
        ],
    )
    def stage(src0, src1, comb, zeros_feat,
              agg0, agg1, acc, idx_a, idx_b, rows_a, rows_b,
              sem_ga, sem_gb, sem_sa, sem_sb):
        cid = lax.axis_index("c")
        sid = lax.axis_index("s")

        _each_slice(sid, rb, rl, lambda r0, sz: pltpu.sync_copy(
            zeros_feat.at[pl.ds(0, sz)], acc.at[pl.ds(r0, sz)]))
        plsc.subcore_barrier()

        def run(src):
            pltpu.sync_copy(comb.at[sid, 0], idx_a)
            pltpu.sync_copy(comb.at[sid, 1], idx_b)
            pltpu.async_copy(src.at[idx_a.at[0]], rows_a, sem_ga)
            pltpu.async_copy(src.at[idx_b.at[0]], rows_b, sem_gb)

            def body(i, carry):
                c0 = 2 * i
                pltpu.make_async_copy(src.at[idx_a.at[0]], rows_a,
                                      sem_ga).wait()
                pltpu.async_copy(rows_a, acc.at[idx_a.at[1]], sem_sa,
                                 add=True)
                pltpu.make_async_copy(src.at[idx_b.at[0]], rows_b,
                                      sem_gb).wait()
                pltpu.async_copy(rows_b, acc.at[idx_b.at[1]], sem_sb,
                                 add=True)

                @pl.when(i < half - 1)
                def _():
                    pltpu.make_async_copy(rows_a, acc.at[idx_a.at[1]],
                                          sem_sa).wait()
                    pltpu.sync_copy(comb.at[sid, c0 + 2], idx_a)
                    pltpu.async_copy(src.at[idx_a.at[0]], rows_a, sem_ga)
                    pltpu.make_async_copy(rows_b, acc.at[idx_b.at[1]],
                                          sem_sb).wait()
                    pltpu.sync_copy(comb.at[sid, c0 + 3], idx_b)
                    pltpu.async_copy(src.at[idx_b.at[0]], rows_b, sem_gb)
                return carry

            lax.fori_loop(0, half, body, 0)
            pltpu.make_async_copy(rows_a, acc.at[idx_a.at[1]], sem_sa).wait()
            pltpu.make_async_copy(rows_b, acc.at[idx_b.at[1]], sem_sb).wait()

        @pl.when(cid == 0)
        def _():
            run(src0)

        @pl.when(cid == 1)
        def _():
            run(src1)

        plsc.subcore_barrier()

        @pl.when(cid == 0)
        def _():
            _each_slice(sid, rb, rl, lambda r0, sz: pltpu.sync_copy(
                acc.at[pl.ds(r0, sz)], agg0.at[pl.ds(r0, sz)]))

        @pl.when(cid == 1)
        def _():
            _each_slice(sid, rb, rl, lambda r0, sz: pltpu.sync_copy(
                acc.at[pl.ds(r0, sz)], agg1.at[pl.ds(r0, sz)]))

    return stage


def _pad_pairs(gidx, sidx, n_edges, trash_row):
    nch = _n_chunks(n_edges)
    per_sub = n_edges // 16
    pad = nch * CHUNK - per_sub
    g = jnp.pad(gidx.reshape(16, per_sub), ((0, 0), (0, pad)),
                constant_values=0).reshape(16, nch, CHUNK)
    s = jnp.pad(sidx.reshape(16, per_sub), ((0, 0), (0, pad)),
                constant_values=trash_row).reshape(16, nch, CHUNK)
    return jnp.stack([g, s], axis=2)


def kernel(x, edge_index, W_lin, b_lin, ln_w, ln_b, W_hg, b_hg, gn_w, gn_b,
           W_skip, b_skip):
    n, in_c = x.shape
    hid = W_lin.shape[0]
    e = edge_index.shape[1]
    seg = NUM_SEGMENTS
    nidx = edge_index[0]
    eidx = edge_index[1]
    comb1 = _pad_pairs(nidx, eidx, e, seg)
    comb2 = _pad_pairs(eidx, nidx, e, n)

    grid = n // ROWS_BLK
    full = lambda shp: pl.BlockSpec(shp, lambda i: (0,) * len(shp))
    row_blk = lambda w: pl.BlockSpec((ROWS_BLK, w), lambda i: (i, 0))

    xw0, xw1, skip = pl.pallas_call(
        _prologue_body,
        grid=(grid,),
        in_specs=[row_blk(in_c), full((hid, in_c)), full((1, hid)),
                  full((1, hid)), full((1, hid)), full((hid, hid)),
                  full((hid, hid)), full((1, hid))],
        out_specs=[row_blk(128), row_blk(128), row_blk(hid)],
        out_shape=[jax.ShapeDtypeStruct((n, 128), jnp.float32),
                   jax.ShapeDtypeStruct((n, 128), jnp.float32),
                   jax.ShapeDtypeStruct((n, hid), jnp.float32)],
    )(x, W_lin, b_lin.reshape(1, hid), ln_w.reshape(1, hid),
      ln_b.reshape(1, hid), W_hg, W_skip, b_skip.reshape(1, hid))

    zrows = max(_sub_rows(seg)[1], _sub_rows(n)[1])
    zeros_feat = jnp.zeros((zrows, 128), jnp.float32)
    ones_feat = jnp.ones((CHUNK, 128), jnp.float32)

    dcnt, bcnt = _make_sc_counts(n, e)(comb1, comb2, zeros_feat, ones_feat)
    agg0, agg1 = _make_sc_stage(seg, e)(xw0, xw1, comb1, zeros_feat)

    out1_0, out1_1 = pl.pallas_call(
        _scale_body,
        grid=(seg // ROWS_BLK,),
        in_specs=[row_blk(128), row_blk(128), row_blk(128)],
        out_specs=[row_blk(128), row_blk(128)],
        out_shape=[jax.ShapeDtypeStruct((seg, 128), jnp.float32),
                   jax.ShapeDtypeStruct((seg, 128), jnp.float32)],
    )(agg0, agg1, bcnt)

    agg2_0, agg2_1 = _make_sc_stage(n, e)(out1_0, out1_1, comb2, zeros_feat)

    out = pl.pallas_call(
        _epilogue_body,
        grid=(grid,),
        in_specs=[row_blk(128), row_blk(128), row_blk(128), full((1, hid)),
                  full((1, hid)), full((1, hid)), row_blk(hid)],
        out_specs=row_blk(hid),
        out_shape=jax.ShapeDtypeStruct((n, hid), jnp.float32),
    )(agg2_0, agg2_1, dcnt, b_hg.reshape(1, hid), gn_w.reshape(1, hid),
      gn_b.reshape(1, hid), skip)

    return out
